# baseline (speedup 1.0000x reference)
import functools
import math

import jax
import jax.numpy as jnp
from jax import lax
from jax.experimental import pallas as pl
from jax.experimental.pallas import tpu as pltpu
from jax.experimental.pallas import tpu_sc as plsc

N = 2048
H = 16
NBUCKETS = 32
MAX_DISTANCE = 128
NV = 16
NS = 8
LW = 4480
LEXT = 4736

_PANELS_PER_TILE = (N // NS) // 32


def _prep_body(table_t_ref, out_ref, acc_ref):
    v = pl.program_id(0)

    @pl.when(v == 0)
    def _compute_line():
        u = lax.broadcasted_iota(jnp.int32, (1, LEXT), 1)
        nv_ = (N - 1) - u
        neg = jnp.where(nv_ < 0, NBUCKETS // 2, 0)
        a = jnp.abs(nv_)
        small = a < (NBUCKETS // 4)
        safe = jnp.maximum(a, 1).astype(jnp.float32)
        t = jnp.log(safe / (NBUCKETS // 4))
        t = t / math.log(MAX_DISTANCE / (NBUCKETS // 4))
        t = t * (NBUCKETS // 2 - NBUCKETS // 4)
        large = (NBUCKETS // 4) + t.astype(jnp.int32)
        large = jnp.minimum(large, NBUCKETS // 2 - 1)
        bucket = neg + jnp.where(small, a, large)

        acc = jnp.zeros((H, LEXT), jnp.float32)
        for b in range(NBUCKETS):
            m = (bucket == b).astype(jnp.float32)
            acc = acc + table_t_ref[:, b:b + 1] * m
        acc_ref[...] = acc

    for vv in range(NV):
        @pl.when(v == vv)
        def _emit(vv=vv):
            for s in range(NS):
                off = 8 * vv + 7 - s
                out_ref[0, :, s, :] = acc_ref[:, off:off + LW]


def _prep(table_t):
    return pl.pallas_call(
        _prep_body,
        grid=(NV,),
        in_specs=[pl.BlockSpec((H, NBUCKETS), lambda v: (0, 0))],
        out_specs=pl.BlockSpec((1, H, NS, LW), lambda v: (v, 0, 0, 0)),
        out_shape=jax.ShapeDtypeStruct((NV, H, NS, LW), jnp.float32),
        scratch_shapes=[pltpu.VMEM((H, LEXT), jnp.float32)],
        compiler_params=pltpu.CompilerParams(
            dimension_semantics=("arbitrary",)),
    )(table_t)


_mesh = plsc.VectorSubcoreMesh(core_axis_name="c", subcore_axis_name="s")


@functools.partial(
    pl.kernel,
    out_type=jax.ShapeDtypeStruct((H, N, N), jnp.float32),
    mesh=_mesh,
    scratch_types=[pltpu.SemaphoreType.DMA],
    compiler_params=pltpu.CompilerParams(use_tc_tiling_on_sc=True),
)
def _expand(skew_hbm, out_hbm, sem):
    cid = lax.axis_index("c")
    sid = lax.axis_index("s")
    w = cid * 16 + sid

    copies = []
    for p in range(_PANELS_PER_TILE):
        it = w * _PANELS_PER_TILE + p
        v = jnp.bitwise_and(255 - it, NV - 1)
        q = pl.multiple_of(2040 - 8 * it - 8 * v, 128)
        row0 = pl.multiple_of(8 * it, 8)
        copies.append(pltpu.async_copy(
            skew_hbm.at[v, :, :, pl.ds(q, N)],
            out_hbm.at[:, pl.ds(row0, NS), :],
            sem,
        ))
    for cp in copies:
        cp.wait()


def kernel(n, relative_attention_bias):
    del n
    table_t = relative_attention_bias.T.astype(jnp.float32)
    skew = _prep(table_t)
    return _expand(skew)


# speedup vs baseline: 87.2316x; 87.2316x over previous
import math

import jax
import jax.numpy as jnp
from jax import lax
from jax.experimental import pallas as pl
from jax.experimental.pallas import tpu as pltpu

N = 2048
H = 16
NBUCKETS = 32
MAX_DISTANCE = 128
NV = 16
NS = 8
LW = 4480
LEXT = 4736

_ROWS_PER_BLOCK = 32
_PANELS_PER_BLOCK = _ROWS_PER_BLOCK // NS
_GRID = N // _ROWS_PER_BLOCK


def _body(table_t_ref, out_ref, skew_ref, acc_ref):
    g = pl.program_id(0)

    @pl.when(g == 0)
    def _build_skew():
        u = lax.broadcasted_iota(jnp.int32, (1, LEXT), 1)
        nv_ = (N - 1) - u
        neg = jnp.where(nv_ < 0, NBUCKETS // 2, 0)
        a = jnp.abs(nv_)
        small = a < (NBUCKETS // 4)
        safe = jnp.maximum(a, 1).astype(jnp.float32)
        t = jnp.log(safe / (NBUCKETS // 4))
        t = t / math.log(MAX_DISTANCE / (NBUCKETS // 4))
        t = t * (NBUCKETS // 2 - NBUCKETS // 4)
        large = (NBUCKETS // 4) + t.astype(jnp.int32)
        large = jnp.minimum(large, NBUCKETS // 2 - 1)
        bucket = neg + jnp.where(small, a, large)

        acc = jnp.zeros((H, LEXT), jnp.float32)
        for b in range(NBUCKETS):
            m = (bucket == b).astype(jnp.float32)
            acc = acc + table_t_ref[:, b:b + 1] * m
        acc_ref[...] = acc
        for v in range(NV):
            for s in range(NS):
                off = 8 * v + 7 - s
                skew_ref[v, :, s, :] = acc_ref[:, off:off + LW]

    for p in range(_PANELS_PER_BLOCK):
        it = g * _PANELS_PER_BLOCK + p
        v = jnp.bitwise_and((N // NS - 1) - it, NV - 1)
        q = pl.multiple_of(2040 - 8 * it - 8 * v, 128)
        out_ref[:, p * NS:(p + 1) * NS, :] = skew_ref[v, :, :, pl.ds(q, N)]


def kernel(n, relative_attention_bias):
    del n
    table_t = relative_attention_bias.T.astype(jnp.float32)
    return pl.pallas_call(
        _body,
        grid=(_GRID,),
        in_specs=[pl.BlockSpec((H, NBUCKETS), lambda g: (0, 0))],
        out_specs=pl.BlockSpec((H, _ROWS_PER_BLOCK, N), lambda g: (0, g, 0)),
        out_shape=jax.ShapeDtypeStruct((H, N, N), jnp.float32),
        scratch_shapes=[
            pltpu.VMEM((NV, H, NS, LW), jnp.float32),
            pltpu.VMEM((H, LEXT), jnp.float32),
        ],
        compiler_params=pltpu.CompilerParams(
            dimension_semantics=("arbitrary",),
            vmem_limit_bytes=100 * 1024 * 1024,
        ),
    )(table_t)
